# baseline (speedup 1.0000x reference)
import functools

import jax
import jax.numpy as jnp
from jax import lax
from jax.experimental import pallas as pl
from jax.experimental.pallas import tpu as pltpu
from jax.experimental.pallas import tpu_sc as plsc

VOCAB = 1000
D = 512
B = 1024
L = 200

SQRT_D = float(512) ** 0.5

NC = 2
NS = 16
BPW = B // NS
LC = 40
NLC = L // LC


def _prep_body(x_ref, tgt_ref, emb_ref, emb2_ref, msrc_ref, mtgt_ref):
    emb2_ref[...] = emb_ref[...] * jnp.float32(SQRT_D)
    xv = x_ref[...]
    msrc_ref[...] = jnp.where(xv == 0, -jnp.inf, 0.0).astype(jnp.float32)
    tv = tgt_ref[...]
    mtgt_ref[...] = jnp.where(tv == 0, -jnp.inf, 0.0).astype(jnp.float32)


_prep = pl.pallas_call(
    _prep_body,
    out_shape=(
        jax.ShapeDtypeStruct((VOCAB, D), jnp.float32),
        jax.ShapeDtypeStruct((B, L), jnp.float32),
        jax.ShapeDtypeStruct((B, L), jnp.float32),
    ),
)


_sc_mesh = plsc.VectorSubcoreMesh(
    core_axis_name="c", subcore_axis_name="s", num_cores=NC, num_subcores=NS
)


@functools.partial(
    pl.kernel,
    out_type=(
        jax.ShapeDtypeStruct((B, L, D), jnp.float32),
        jax.ShapeDtypeStruct((B, L, D), jnp.float32),
    ),
    mesh=_sc_mesh,
    compiler_params=pltpu.CompilerParams(use_tc_tiling_on_sc=False),
    scratch_types=[
        pltpu.VMEM((BPW, L), jnp.int32),
        pltpu.VMEM((LC, D), jnp.float32),
        pltpu.VMEM((LC, D), jnp.float32),
        pltpu.SemaphoreType.DMA,
    ],
)
def _sc_embed(emb_hbm, x_hbm, tgt_hbm, ps_hbm, pt_hbm, xe_hbm, ye_hbm,
              idx_v, pos_v, rows_v, sem):
    cid = lax.axis_index("c")
    sid = lax.axis_index("s")
    b0 = sid * BPW

    def run(tok_hbm, pos_hbm, out_hbm):
        pltpu.sync_copy(tok_hbm.at[pl.ds(b0, BPW)], idx_v)
        for lc in range(NLC):
            l0 = lc * LC
            pltpu.sync_copy(pos_hbm.at[pl.ds(l0, LC)], pos_v)

            def b_body(b, carry):
                pltpu.async_copy(
                    emb_hbm.at[idx_v.at[b, pl.ds(l0, LC)]], rows_v, sem
                ).wait()

                def r_body(r, c2):
                    for k in range(D // 16):
                        sl = pl.ds(k * 16, 16)
                        plsc.addupdate(rows_v.at[r, sl], pos_v[r, sl])
                    return c2

                lax.fori_loop(0, LC, r_body, 0)
                pltpu.sync_copy(rows_v, out_hbm.at[b0 + b, pl.ds(l0, LC)])
                return carry

            lax.fori_loop(0, BPW, b_body, 0)

    @pl.when(cid == 0)
    def _():
        run(x_hbm, ps_hbm, xe_hbm)

    @pl.when(cid == 1)
    def _():
        run(tgt_hbm, pt_hbm, ye_hbm)


def kernel(x, y, embedding, pos_src, pos_tgt):
    tgt = y[:, :-1]
    labels = y[:, 1:]
    emb2, mask_src, mask_tgt = _prep(x, tgt, embedding)
    xe, ye = _sc_embed(emb2, x, tgt, pos_src[:L], pos_tgt[:L])
    return (xe, ye, mask_src, mask_tgt, mask_src, labels)


# speedup vs baseline: 1.0081x; 1.0081x over previous
import functools

import jax
import jax.numpy as jnp
from jax import lax
from jax.experimental import pallas as pl
from jax.experimental.pallas import tpu as pltpu
from jax.experimental.pallas import tpu_sc as plsc

VOCAB = 1000
D = 512
B = 1024
L = 200

SQRT_D = float(512) ** 0.5

NC = 2
NS = 16
NW = NC * NS
BT = 2 * B
BPW = BT // NW
LC = 40
NLC = L // LC
NBUF = 4
LOOKAHEAD = 2


def _prep_body(x_ref, tgt_ref, emb_ref, emb2_ref, msrc_ref, mtgt_ref):
    emb2_ref[...] = emb_ref[...] * jnp.float32(SQRT_D)
    xv = x_ref[...]
    msrc_ref[...] = jnp.where(xv == 0, -jnp.inf, 0.0).astype(jnp.float32)
    tv = tgt_ref[...]
    mtgt_ref[...] = jnp.where(tv == 0, -jnp.inf, 0.0).astype(jnp.float32)


_prep = pl.pallas_call(
    _prep_body,
    out_shape=(
        jax.ShapeDtypeStruct((VOCAB, D), jnp.float32),
        jax.ShapeDtypeStruct((B, L), jnp.float32),
        jax.ShapeDtypeStruct((B, L), jnp.float32),
    ),
)


_sc_mesh = plsc.VectorSubcoreMesh(
    core_axis_name="c", subcore_axis_name="s", num_cores=NC, num_subcores=NS
)


@functools.partial(
    pl.kernel,
    out_type=jax.ShapeDtypeStruct((BT, L, D), jnp.float32),
    mesh=_sc_mesh,
    compiler_params=pltpu.CompilerParams(use_tc_tiling_on_sc=False),
    scratch_types=[
        pltpu.VMEM((BPW, L), jnp.int32),
        pltpu.VMEM((LC, D), jnp.float32),
        pltpu.VMEM((NBUF, LC, D), jnp.float32),
    ]
    + [pltpu.SemaphoreType.DMA] * (2 * NBUF),
)
def _sc_embed(emb_hbm, tok_hbm, pos_hbm, out_hbm,
              idx_v, pos_v, rows_v,
              g0, g1, g2, g3, w0, w1, w2, w3):
    gsem = [g0, g1, g2, g3]
    wsem = [w0, w1, w2, w3]
    cid = lax.axis_index("c")
    sid = lax.axis_index("s")
    wid = cid * NS + sid
    b0 = wid * BPW
    t = cid

    pltpu.sync_copy(tok_hbm.at[pl.ds(b0, BPW)], idx_v)

    def gather_desc(bb, j, l0):
        return pltpu.make_async_copy(
            emb_hbm.at[idx_v.at[bb, pl.ds(l0, LC)]], rows_v.at[j], gsem[j]
        )

    def write_desc(bb, j, l0):
        return pltpu.make_async_copy(
            rows_v.at[j], out_hbm.at[b0 + bb, pl.ds(l0, LC)], wsem[j]
        )

    def lc_body(lc, carry):
        l0 = lc * LC
        pltpu.sync_copy(pos_hbm.at[t, pl.ds(l0, LC)], pos_v)

        for j in range(NBUF):
            gather_desc(j, j, l0).start()

        def g_body(g, c1):
            base = g * NBUF
            for j in range(NBUF):
                bb = base + j
                gather_desc(bb, j, l0).wait()

                def r_body(r, c2):
                    for k in range(D // 16):
                        sl = pl.ds(k * 16, 16)
                        plsc.addupdate(rows_v.at[j, r, sl], pos_v[r, sl])
                    return c2

                lax.fori_loop(0, LC, r_body, 0, unroll=False)
                write_desc(bb, j, l0).start()

            for j in range(NBUF):
                bb = base + j
                write_desc(bb, j, l0).wait()
                nxt = base + NBUF + j

                @pl.when(nxt < BPW)
                def _():
                    gather_desc(nxt, j, l0).start()

            return c1

        lax.fori_loop(0, BPW // NBUF, g_body, 0, unroll=False)
        return carry

    lax.fori_loop(0, NLC, lc_body, 0, unroll=False)


def kernel(x, y, embedding, pos_src, pos_tgt):
    tgt = y[:, :-1]
    labels = y[:, 1:]
    emb2, mask_src, mask_tgt = _prep(x, tgt, embedding)
    tok = jnp.concatenate([x, tgt], axis=0)
    pos = jnp.stack([pos_src[:L], pos_tgt[:L]], axis=0)
    out = _sc_embed(emb2, tok, pos)
    return (out[:B], out[B:], mask_src, mask_tgt, mask_src, labels)


# speedup vs baseline: 1.4492x; 1.4375x over previous
import functools

import jax
import jax.numpy as jnp
from jax import lax
from jax.experimental import pallas as pl
from jax.experimental.pallas import tpu as pltpu
from jax.experimental.pallas import tpu_sc as plsc

VOCAB = 1000
D = 512
B = 1024
L = 200

SQRT_D = float(512) ** 0.5

NC = 2
NS = 16
NW = NC * NS
BPW = B // NW
LC = 40
NLC = L // LC
NBUF = 4


def _prep_body(x_ref, tgt_ref, emb_ref, emb2_ref, msrc_ref, mtgt_ref):
    emb2_ref[...] = emb_ref[...] * jnp.float32(SQRT_D)
    xv = x_ref[...]
    msrc_ref[...] = jnp.where(xv == 0, -jnp.inf, 0.0).astype(jnp.float32)
    tv = tgt_ref[...]
    mtgt_ref[...] = jnp.where(tv == 0, -jnp.inf, 0.0).astype(jnp.float32)


_prep = pl.pallas_call(
    _prep_body,
    out_shape=(
        jax.ShapeDtypeStruct((VOCAB, D), jnp.float32),
        jax.ShapeDtypeStruct((B, L), jnp.float32),
        jax.ShapeDtypeStruct((B, L), jnp.float32),
    ),
)


_sc_mesh = plsc.VectorSubcoreMesh(
    core_axis_name="c", subcore_axis_name="s", num_cores=NC, num_subcores=NS
)


@functools.partial(
    pl.kernel,
    out_type=jax.ShapeDtypeStruct((B, L, D), jnp.float32),
    mesh=_sc_mesh,
    compiler_params=pltpu.CompilerParams(use_tc_tiling_on_sc=False),
    scratch_types=[
        pltpu.VMEM((BPW, L), jnp.int32),
        pltpu.VMEM((LC, D), jnp.float32),
        pltpu.VMEM((NBUF, LC, D), jnp.float32),
    ]
    + [pltpu.SemaphoreType.DMA] * (2 * NBUF),
)
def _sc_embed(emb_hbm, tok_hbm, pos_hbm, out_hbm,
              idx_v, pos_v, rows_v,
              g0, g1, g2, g3, w0, w1, w2, w3):
    gsem = [g0, g1, g2, g3]
    wsem = [w0, w1, w2, w3]
    cid = lax.axis_index("c")
    sid = lax.axis_index("s")
    wid = cid * NS + sid
    b0 = wid * BPW

    pltpu.sync_copy(tok_hbm.at[pl.ds(b0, BPW)], idx_v)

    def gather_desc(bb, j, l0):
        return pltpu.make_async_copy(
            emb_hbm.at[idx_v.at[bb, pl.ds(l0, LC)]], rows_v.at[j], gsem[j]
        )

    def write_desc(bb, j, l0):
        return pltpu.make_async_copy(
            rows_v.at[j], out_hbm.at[b0 + bb, pl.ds(l0, LC)], wsem[j]
        )

    def lc_body(lc, carry):
        l0 = lc * LC
        pltpu.sync_copy(pos_hbm.at[pl.ds(l0, LC)], pos_v)

        for j in range(NBUF):
            gather_desc(j, j, l0).start()

        def g_body(g, c1):
            base = g * NBUF
            for j in range(NBUF):
                bb = base + j
                gather_desc(bb, j, l0).wait()

                def r_body(r, c2):
                    for k in range(D // 16):
                        sl = pl.ds(k * 16, 16)
                        plsc.addupdate(rows_v.at[j, r, sl], pos_v[r, sl])
                    return c2

                lax.fori_loop(0, LC, r_body, 0, unroll=False)
                write_desc(bb, j, l0).start()

            for j in range(NBUF):
                bb = base + j
                write_desc(bb, j, l0).wait()
                nxt = base + NBUF + j

                @pl.when(nxt < BPW)
                def _():
                    gather_desc(nxt, j, l0).start()

            return c1

        lax.fori_loop(0, BPW // NBUF, g_body, 0, unroll=False)
        return carry

    lax.fori_loop(0, NLC, lc_body, 0, unroll=False)


def kernel(x, y, embedding, pos_src, pos_tgt):
    tgt = y[:, :-1]
    labels = y[:, 1:]
    emb2, mask_src, mask_tgt = _prep(x, tgt, embedding)
    xe = _sc_embed(emb2, x, pos_src[:L])
    ye = _sc_embed(emb2, tgt, pos_tgt[:L])
    return (xe, ye, mask_src, mask_tgt, mask_src, labels)


# speedup vs baseline: 1.5077x; 1.0403x over previous
import functools

import jax
import jax.numpy as jnp
from jax import lax
from jax.experimental import pallas as pl
from jax.experimental.pallas import tpu as pltpu
from jax.experimental.pallas import tpu_sc as plsc

VOCAB = 1000
D = 512
B = 1024
L = 200

SQRT_D = float(512) ** 0.5

NC = 2
NS = 16
NW = NC * NS
BPW = B // NW
LC = 40
NLC = L // LC
NBUF = 4


def _prep_body(x_ref, tgt_ref, emb_ref, emb2_ref, msrc_ref, mtgt_ref):
    emb2_ref[...] = emb_ref[...] * jnp.float32(SQRT_D)
    xv = x_ref[...]
    msrc_ref[...] = jnp.where(xv == 0, -jnp.inf, 0.0).astype(jnp.float32)
    tv = tgt_ref[...]
    mtgt_ref[...] = jnp.where(tv == 0, -jnp.inf, 0.0).astype(jnp.float32)


_prep = pl.pallas_call(
    _prep_body,
    out_shape=(
        jax.ShapeDtypeStruct((VOCAB, D), jnp.float32),
        jax.ShapeDtypeStruct((B, L), jnp.float32),
        jax.ShapeDtypeStruct((B, L), jnp.float32),
    ),
)


_sc_mesh = plsc.VectorSubcoreMesh(
    core_axis_name="c", subcore_axis_name="s", num_cores=NC, num_subcores=NS
)


@functools.partial(
    pl.kernel,
    out_type=jax.ShapeDtypeStruct((B, L, D), jnp.float32),
    mesh=_sc_mesh,
    compiler_params=pltpu.CompilerParams(use_tc_tiling_on_sc=False),
    scratch_types=[
        pltpu.VMEM((BPW, L), jnp.int32),
        pltpu.VMEM((LC, D), jnp.float32),
        pltpu.VMEM((NBUF, LC, D), jnp.float32),
    ]
    + [pltpu.SemaphoreType.DMA] * (2 * NBUF),
)
def _sc_embed(emb_hbm, tok_hbm, pos_hbm, out_hbm,
              idx_v, pos_v, rows_v,
              g0, g1, g2, g3, w0, w1, w2, w3):
    gsem = [g0, g1, g2, g3]
    wsem = [w0, w1, w2, w3]
    cid = lax.axis_index("c")
    sid = lax.axis_index("s")
    wid = cid * NS + sid
    b0 = wid * BPW

    pltpu.sync_copy(tok_hbm.at[pl.ds(b0, BPW)], idx_v)

    def gather_desc(bb, j, l0):
        return pltpu.make_async_copy(
            emb_hbm.at[idx_v.at[bb, pl.ds(l0, LC)]], rows_v.at[j], gsem[j]
        )

    def write_desc(bb, j, l0):
        return pltpu.make_async_copy(
            rows_v.at[j], out_hbm.at[b0 + bb, pl.ds(l0, LC)], wsem[j]
        )

    def lc_body(lc, carry):
        l0 = lc * LC
        pltpu.sync_copy(pos_hbm.at[pl.ds(l0, LC)], pos_v)

        for j in range(2):
            gather_desc(j, j, l0).start()

        def g_body(g, c1):
            base = g * NBUF
            for j in range(NBUF):
                bb = base + j
                gather_desc(bb, j, l0).wait()

                def r_body(r, c2):
                    for k in range(D // 16):
                        sl = pl.ds(k * 16, 16)
                        plsc.addupdate(rows_v.at[j, r, sl], pos_v[r, sl])
                    return c2

                lax.fori_loop(0, LC, r_body, 0, unroll=False)
                write_desc(bb, j, l0).start()

                nxt = bb + 2
                jn = (j + 2) % NBUF

                @pl.when(nxt < BPW)
                def _():
                    prev = nxt - NBUF

                    @pl.when(prev >= 0)
                    def _():
                        write_desc(prev, jn, l0).wait()

                    gather_desc(nxt, jn, l0).start()

            return c1

        lax.fori_loop(0, BPW // NBUF, g_body, 0, unroll=False)
        write_desc(BPW - 2, (BPW - 2) % NBUF, l0).wait()
        write_desc(BPW - 1, (BPW - 1) % NBUF, l0).wait()
        return carry

    lax.fori_loop(0, NLC, lc_body, 0, unroll=False)


def kernel(x, y, embedding, pos_src, pos_tgt):
    tgt = y[:, :-1]
    labels = y[:, 1:]
    emb2, mask_src, mask_tgt = _prep(x, tgt, embedding)
    xe = _sc_embed(emb2, x, pos_src[:L])
    ye = _sc_embed(emb2, tgt, pos_tgt[:L])
    return (xe, ye, mask_src, mask_tgt, mask_src, labels)


# speedup vs baseline: 1.6732x; 1.1098x over previous
import functools

import jax
import jax.numpy as jnp
from jax import lax
from jax.experimental import pallas as pl
from jax.experimental.pallas import tpu as pltpu
from jax.experimental.pallas import tpu_sc as plsc

VOCAB = 1000
D = 512
B = 1024
L = 200

SQRT_D = float(512) ** 0.5

NC = 2
NS = 16
NW = NC * NS
BPW = B // NW
LC = 40
NLC = L // LC
NBUF = 4


def _prep_body(x_ref, tgt_ref, emb_ref, emb2_ref, hi_ref, lo_ref,
               msrc_ref, mtgt_ref):
    emb2 = emb_ref[...] * jnp.float32(SQRT_D)
    emb2_ref[...] = emb2
    hi = emb2.astype(jnp.bfloat16)
    hi_ref[...] = hi
    lo_ref[...] = (emb2 - hi.astype(jnp.float32)).astype(jnp.bfloat16)
    xv = x_ref[...]
    msrc_ref[...] = jnp.where(xv == 0, -jnp.inf, 0.0).astype(jnp.float32)
    tv = tgt_ref[...]
    mtgt_ref[...] = jnp.where(tv == 0, -jnp.inf, 0.0).astype(jnp.float32)


_prep = pl.pallas_call(
    _prep_body,
    out_shape=(
        jax.ShapeDtypeStruct((VOCAB, D), jnp.float32),
        jax.ShapeDtypeStruct((VOCAB, D), jnp.bfloat16),
        jax.ShapeDtypeStruct((VOCAB, D), jnp.bfloat16),
        jax.ShapeDtypeStruct((B, L), jnp.float32),
        jax.ShapeDtypeStruct((B, L), jnp.float32),
    ),
)


def _ye_body(tok_ref, hi_ref, lo_ref, pos_ref, out_ref):
    toks = tok_ref[0, 0, :]
    oh = (
        lax.broadcasted_iota(jnp.int32, (L, VOCAB), 1) == toks[:, None]
    ).astype(jnp.bfloat16)
    dn = (((1,), (0,)), ((), ()))
    acc = lax.dot_general(oh, hi_ref[...], dn,
                          preferred_element_type=jnp.float32)
    acc = acc + lax.dot_general(oh, lo_ref[...], dn,
                                preferred_element_type=jnp.float32)
    out_ref[0] = acc + pos_ref[...]


_ye_mxu = pl.pallas_call(
    _ye_body,
    grid=(B,),
    in_specs=[
        pl.BlockSpec((1, 1, L), lambda i: (i, 0, 0)),
        pl.BlockSpec((VOCAB, D), lambda i: (0, 0)),
        pl.BlockSpec((VOCAB, D), lambda i: (0, 0)),
        pl.BlockSpec((L, D), lambda i: (0, 0)),
    ],
    out_specs=pl.BlockSpec((1, L, D), lambda i: (i, 0, 0)),
    out_shape=jax.ShapeDtypeStruct((B, L, D), jnp.float32),
)


_sc_mesh = plsc.VectorSubcoreMesh(
    core_axis_name="c", subcore_axis_name="s", num_cores=NC, num_subcores=NS
)


@functools.partial(
    pl.kernel,
    out_type=jax.ShapeDtypeStruct((B, L, D), jnp.float32),
    mesh=_sc_mesh,
    compiler_params=pltpu.CompilerParams(use_tc_tiling_on_sc=False),
    scratch_types=[
        pltpu.VMEM((BPW, L), jnp.int32),
        pltpu.VMEM((LC, D), jnp.float32),
        pltpu.VMEM((NBUF, LC, D), jnp.float32),
    ]
    + [pltpu.SemaphoreType.DMA] * (2 * NBUF),
)
def _sc_embed(emb_hbm, tok_hbm, pos_hbm, out_hbm,
              idx_v, pos_v, rows_v,
              g0, g1, g2, g3, w0, w1, w2, w3):
    gsem = [g0, g1, g2, g3]
    wsem = [w0, w1, w2, w3]
    cid = lax.axis_index("c")
    sid = lax.axis_index("s")
    wid = cid * NS + sid
    b0 = wid * BPW

    pltpu.sync_copy(tok_hbm.at[pl.ds(b0, BPW)], idx_v)

    def gather_desc(bb, j, l0):
        return pltpu.make_async_copy(
            emb_hbm.at[idx_v.at[bb, pl.ds(l0, LC)]], rows_v.at[j], gsem[j]
        )

    def write_desc(bb, j, l0):
        return pltpu.make_async_copy(
            rows_v.at[j], out_hbm.at[b0 + bb, pl.ds(l0, LC)], wsem[j]
        )

    def lc_body(lc, carry):
        l0 = lc * LC
        pltpu.sync_copy(pos_hbm.at[pl.ds(l0, LC)], pos_v)

        for j in range(2):
            gather_desc(j, j, l0).start()

        def g_body(g, c1):
            base = g * NBUF
            for j in range(NBUF):
                bb = base + j
                gather_desc(bb, j, l0).wait()

                def r_body(r, c2):
                    for k in range(D // 16):
                        sl = pl.ds(k * 16, 16)
                        plsc.addupdate(rows_v.at[j, r, sl], pos_v[r, sl])
                    return c2

                lax.fori_loop(0, LC, r_body, 0, unroll=False)
                write_desc(bb, j, l0).start()

                nxt = bb + 2
                jn = (j + 2) % NBUF

                @pl.when(nxt < BPW)
                def _():
                    prev = nxt - NBUF

                    @pl.when(prev >= 0)
                    def _():
                        write_desc(prev, jn, l0).wait()

                    gather_desc(nxt, jn, l0).start()

            return c1

        lax.fori_loop(0, BPW // NBUF, g_body, 0, unroll=False)
        write_desc(BPW - 2, (BPW - 2) % NBUF, l0).wait()
        write_desc(BPW - 1, (BPW - 1) % NBUF, l0).wait()
        return carry

    lax.fori_loop(0, NLC, lc_body, 0, unroll=False)


def kernel(x, y, embedding, pos_src, pos_tgt):
    tgt = y[:, :-1]
    labels = y[:, 1:]
    emb2, hi, lo, mask_src, mask_tgt = _prep(x, tgt, embedding)
    xe = _sc_embed(emb2, x, pos_src[:L])
    ye = _ye_mxu(tgt.reshape(B, 1, L), hi, lo, pos_tgt[:L])
    return (xe, ye, mask_src, mask_tgt, mask_src, labels)


# speedup vs baseline: 2.2038x; 1.3171x over previous
import functools

import jax
import jax.numpy as jnp
from jax import lax
from jax.experimental import pallas as pl
from jax.experimental.pallas import tpu as pltpu
from jax.experimental.pallas import tpu_sc as plsc

VOCAB = 1000
D = 512
B = 1024
L = 200

SQRT_D = float(512) ** 0.5

NC = 2
NS = 16
NW = NC * NS
BPW = B // NW
LC = 40
NLC = L // LC
NBUF = 4


def _prep_body(x_ref, tgt_ref, emb_ref, emb2_ref, hi_ref, lo_ref,
               msrc_ref, mtgt_ref):
    emb2 = emb_ref[...] * jnp.float32(SQRT_D)
    emb2_ref[...] = emb2
    hi = emb2.astype(jnp.bfloat16)
    hi_ref[...] = hi
    lo_ref[...] = (emb2 - hi.astype(jnp.float32)).astype(jnp.bfloat16)
    xv = x_ref[...]
    msrc_ref[...] = jnp.where(xv == 0, -jnp.inf, 0.0).astype(jnp.float32)
    tv = tgt_ref[...]
    mtgt_ref[...] = jnp.where(tv == 0, -jnp.inf, 0.0).astype(jnp.float32)


_prep = pl.pallas_call(
    _prep_body,
    out_shape=(
        jax.ShapeDtypeStruct((VOCAB, D), jnp.float32),
        jax.ShapeDtypeStruct((VOCAB, D), jnp.bfloat16),
        jax.ShapeDtypeStruct((VOCAB, D), jnp.bfloat16),
        jax.ShapeDtypeStruct((B, L), jnp.float32),
        jax.ShapeDtypeStruct((B, L), jnp.float32),
    ),
)


BS = 8


def _ye_body(tok_ref, hi_ref, lo_ref, pos_ref, out_ref):
    iota = lax.broadcasted_iota(jnp.int32, (L, VOCAB), 1)
    dn = (((1,), (0,)), ((), ()))
    pos = pos_ref[...]
    for i in range(BS):
        toks = tok_ref[i, 0, :]
        oh = (iota == toks[:, None]).astype(jnp.bfloat16)
        acc = lax.dot_general(oh, hi_ref[...], dn,
                              preferred_element_type=jnp.float32)
        acc = acc + lax.dot_general(oh, lo_ref[...], dn,
                                    preferred_element_type=jnp.float32)
        out_ref[i] = acc + pos


_ye_mxu = pl.pallas_call(
    _ye_body,
    grid=(B // BS,),
    in_specs=[
        pl.BlockSpec((BS, 1, L), lambda i: (i, 0, 0)),
        pl.BlockSpec((VOCAB, D), lambda i: (0, 0)),
        pl.BlockSpec((VOCAB, D), lambda i: (0, 0)),
        pl.BlockSpec((L, D), lambda i: (0, 0)),
    ],
    out_specs=pl.BlockSpec((BS, L, D), lambda i: (i, 0, 0)),
    out_shape=jax.ShapeDtypeStruct((B, L, D), jnp.float32),
)


_sc_mesh = plsc.VectorSubcoreMesh(
    core_axis_name="c", subcore_axis_name="s", num_cores=NC, num_subcores=NS
)


@functools.partial(
    pl.kernel,
    out_type=jax.ShapeDtypeStruct((B, L, D), jnp.float32),
    mesh=_sc_mesh,
    compiler_params=pltpu.CompilerParams(use_tc_tiling_on_sc=False),
    scratch_types=[
        pltpu.VMEM((BPW, L), jnp.int32),
        pltpu.VMEM((LC, D), jnp.float32),
        pltpu.VMEM((NBUF, LC, D), jnp.float32),
    ]
    + [pltpu.SemaphoreType.DMA] * (2 * NBUF),
)
def _sc_embed(emb_hbm, tok_hbm, pos_hbm, out_hbm,
              idx_v, pos_v, rows_v,
              g0, g1, g2, g3, w0, w1, w2, w3):
    gsem = [g0, g1, g2, g3]
    wsem = [w0, w1, w2, w3]
    cid = lax.axis_index("c")
    sid = lax.axis_index("s")
    wid = cid * NS + sid
    b0 = wid * BPW

    pltpu.sync_copy(tok_hbm.at[pl.ds(b0, BPW)], idx_v)

    def gather_desc(bb, j, l0):
        return pltpu.make_async_copy(
            emb_hbm.at[idx_v.at[bb, pl.ds(l0, LC)]], rows_v.at[j], gsem[j]
        )

    def write_desc(bb, j, l0):
        return pltpu.make_async_copy(
            rows_v.at[j], out_hbm.at[b0 + bb, pl.ds(l0, LC)], wsem[j]
        )

    def lc_body(lc, carry):
        l0 = lc * LC
        pltpu.sync_copy(pos_hbm.at[pl.ds(l0, LC)], pos_v)

        for j in range(2):
            gather_desc(j, j, l0).start()

        def g_body(g, c1):
            base = g * NBUF
            for j in range(NBUF):
                bb = base + j
                gather_desc(bb, j, l0).wait()

                def r_body(r, c2):
                    for k in range(D // 16):
                        sl = pl.ds(k * 16, 16)
                        plsc.addupdate(rows_v.at[j, r, sl], pos_v[r, sl])
                    return c2

                lax.fori_loop(0, LC, r_body, 0, unroll=False)
                write_desc(bb, j, l0).start()

                nxt = bb + 2
                jn = (j + 2) % NBUF

                @pl.when(nxt < BPW)
                def _():
                    prev = nxt - NBUF

                    @pl.when(prev >= 0)
                    def _():
                        write_desc(prev, jn, l0).wait()

                    gather_desc(nxt, jn, l0).start()

            return c1

        lax.fori_loop(0, BPW // NBUF, g_body, 0, unroll=False)
        write_desc(BPW - 2, (BPW - 2) % NBUF, l0).wait()
        write_desc(BPW - 1, (BPW - 1) % NBUF, l0).wait()
        return carry

    lax.fori_loop(0, NLC, lc_body, 0, unroll=False)


def kernel(x, y, embedding, pos_src, pos_tgt):
    tgt = y[:, :-1]
    labels = y[:, 1:]
    emb2, hi, lo, mask_src, mask_tgt = _prep(x, tgt, embedding)
    xe = _sc_embed(emb2, x, pos_src[:L])
    ye = _ye_mxu(tgt.reshape(B, 1, L), hi, lo, pos_tgt[:L])
    return (xe, ye, mask_src, mask_tgt, mask_src, labels)


# speedup vs baseline: 2.2092x; 1.0025x over previous
import functools

import jax
import jax.numpy as jnp
from jax import lax
from jax.experimental import pallas as pl
from jax.experimental.pallas import tpu as pltpu
from jax.experimental.pallas import tpu_sc as plsc

VOCAB = 1000
D = 512
B = 1024
L = 200

SQRT_D = float(512) ** 0.5

NC = 2
NS = 16
NW = NC * NS
BPW = B // NW
LC = 40
NLC = L // LC
NBUF = 4


def _prep_body(x_ref, tgt_ref, emb_ref, emb2_ref, hi_ref, lo_ref,
               msrc_ref, mtgt_ref):
    emb2 = emb_ref[...] * jnp.float32(SQRT_D)
    emb2_ref[...] = emb2
    hi = emb2.astype(jnp.bfloat16)
    hi_ref[...] = hi
    lo_ref[...] = (emb2 - hi.astype(jnp.float32)).astype(jnp.bfloat16)
    xv = x_ref[...]
    msrc_ref[...] = jnp.where(xv == 0, -jnp.inf, 0.0).astype(jnp.float32)
    tv = tgt_ref[...]
    mtgt_ref[...] = jnp.where(tv == 0, -jnp.inf, 0.0).astype(jnp.float32)


_prep = pl.pallas_call(
    _prep_body,
    out_shape=(
        jax.ShapeDtypeStruct((VOCAB, D), jnp.float32),
        jax.ShapeDtypeStruct((VOCAB, D), jnp.bfloat16),
        jax.ShapeDtypeStruct((VOCAB, D), jnp.bfloat16),
        jax.ShapeDtypeStruct((B, L), jnp.float32),
        jax.ShapeDtypeStruct((B, L), jnp.float32),
    ),
)


BS = 8


def _ye_body(tok_ref, hi_ref, lo_ref, pos_ref, out_ref):
    iota = lax.broadcasted_iota(jnp.int32, (L, VOCAB), 1)
    dn = (((1,), (0,)), ((), ()))
    pos = pos_ref[...]
    for i in range(BS):
        toks = tok_ref[0, i, :]
        oh = (iota == toks[:, None]).astype(jnp.bfloat16)
        acc = lax.dot_general(oh, hi_ref[...], dn,
                              preferred_element_type=jnp.float32)
        acc = acc + lax.dot_general(oh, lo_ref[...], dn,
                                    preferred_element_type=jnp.float32)
        out_ref[i] = acc + pos


_ye_mxu = pl.pallas_call(
    _ye_body,
    grid=(B // BS,),
    in_specs=[
        pl.BlockSpec((1, BS, L), lambda i: (i, 0, 0)),
        pl.BlockSpec((VOCAB, D), lambda i: (0, 0)),
        pl.BlockSpec((VOCAB, D), lambda i: (0, 0)),
        pl.BlockSpec((L, D), lambda i: (0, 0)),
    ],
    out_specs=pl.BlockSpec((BS, L, D), lambda i: (i, 0, 0)),
    out_shape=jax.ShapeDtypeStruct((B, L, D), jnp.float32),
)


_sc_mesh = plsc.VectorSubcoreMesh(
    core_axis_name="c", subcore_axis_name="s", num_cores=NC, num_subcores=NS
)


@functools.partial(
    pl.kernel,
    out_type=jax.ShapeDtypeStruct((B, L, D), jnp.float32),
    mesh=_sc_mesh,
    compiler_params=pltpu.CompilerParams(use_tc_tiling_on_sc=False),
    scratch_types=[
        pltpu.VMEM((BPW, L), jnp.int32),
        pltpu.VMEM((LC, D), jnp.float32),
        pltpu.VMEM((NBUF, LC, D), jnp.float32),
    ]
    + [pltpu.SemaphoreType.DMA] * (2 * NBUF),
)
def _sc_embed(emb_hbm, tok_hbm, pos_hbm, out_hbm,
              idx_v, pos_v, rows_v,
              g0, g1, g2, g3, w0, w1, w2, w3):
    gsem = [g0, g1, g2, g3]
    wsem = [w0, w1, w2, w3]
    cid = lax.axis_index("c")
    sid = lax.axis_index("s")
    wid = cid * NS + sid
    b0 = wid * BPW

    pltpu.sync_copy(tok_hbm.at[pl.ds(b0, BPW)], idx_v)

    def gather_desc(bb, j, l0):
        return pltpu.make_async_copy(
            emb_hbm.at[idx_v.at[bb, pl.ds(l0, LC)]], rows_v.at[j], gsem[j]
        )

    def write_desc(bb, j, l0):
        return pltpu.make_async_copy(
            rows_v.at[j], out_hbm.at[b0 + bb, pl.ds(l0, LC)], wsem[j]
        )

    def lc_body(lc, carry):
        l0 = lc * LC
        pltpu.sync_copy(pos_hbm.at[pl.ds(l0, LC)], pos_v)

        for j in range(2):
            gather_desc(j, j, l0).start()

        def g_body(g, c1):
            base = g * NBUF
            for j in range(NBUF):
                bb = base + j
                gather_desc(bb, j, l0).wait()

                def r_body(r, c2):
                    for k in range(D // 16):
                        sl = pl.ds(k * 16, 16)
                        plsc.addupdate(rows_v.at[j, r, sl], pos_v[r, sl])
                    return c2

                lax.fori_loop(0, LC, r_body, 0, unroll=False)
                write_desc(bb, j, l0).start()

                nxt = bb + 2
                jn = (j + 2) % NBUF

                @pl.when(nxt < BPW)
                def _():
                    prev = nxt - NBUF

                    @pl.when(prev >= 0)
                    def _():
                        write_desc(prev, jn, l0).wait()

                    gather_desc(nxt, jn, l0).start()

            return c1

        lax.fori_loop(0, BPW // NBUF, g_body, 0, unroll=False)
        write_desc(BPW - 2, (BPW - 2) % NBUF, l0).wait()
        write_desc(BPW - 1, (BPW - 1) % NBUF, l0).wait()
        return carry

    lax.fori_loop(0, NLC, lc_body, 0, unroll=False)


def kernel(x, y, embedding, pos_src, pos_tgt):
    tgt = y[:, :-1]
    labels = y[:, 1:]
    emb2, hi, lo, mask_src, mask_tgt = _prep(x, tgt, embedding)
    xe = _sc_embed(emb2, x, pos_src[:L])
    ye = _ye_mxu(tgt.reshape(B // BS, BS, L), hi, lo, pos_tgt[:L])
    return (xe, ye, mask_src, mask_tgt, mask_src, labels)
